# baseline (speedup 1.0000x reference)
import dataclasses
import functools

import jax
import jax.numpy as jnp
from jax import lax
from jax.experimental import pallas as pl
from jax.experimental.pallas import tpu as pltpu
from jax.experimental.pallas import tpu_sc as plsc

DIM = 32
PADW = 128
R = 2
LANES = 16
NTEC = 32
SPLITS = ((0, 128), (128, 72))


def _zero_padding_rows(idx_row, out_row):
    zeros = jnp.zeros((LANES,), jnp.float32)
    for off in list(range(0, 192, LANES)) + [200 - LANES]:
        v = idx_row[pl.ds(off, LANES)]
        is_pad = v == 0

        @pl.when(jnp.any(is_pad))
        def _():
            rows = jnp.arange(LANES, dtype=jnp.int32) + off

            @pl.loop(0, DIM)
            def _(c):
                cols = jnp.full((LANES,), 0, jnp.int32) + c
                plsc.store_scatter(out_row, [rows, cols], zeros, mask=is_pad)


def _make_kernel(n_rows, n_cols):
    mesh = plsc.VectorSubcoreMesh(core_axis_name="c", subcore_axis_name="s")
    out_sds = jax.ShapeDtypeStruct((n_rows, n_cols, DIM), jnp.float32)
    rows_per_tec = n_rows // NTEC
    n_ch = rows_per_tec // R

    cp = pltpu.CompilerParams()
    fields = pltpu.CompilerParams.__dataclass_fields__
    if "needs_layout_passes" in fields:
        cp = dataclasses.replace(cp, needs_layout_passes=False)
    if "use_tc_tiling_on_sc" in fields:
        cp = dataclasses.replace(cp, use_tc_tiling_on_sc=False)

    @functools.partial(
        pl.kernel,
        out_type=out_sds,
        mesh=mesh,
        compiler_params=cp,
        scratch_types=[
            pltpu.VMEM((R, n_cols), jnp.int32),
            pltpu.VMEM((R, n_cols), jnp.int32),
            pltpu.VMEM((R, n_cols, PADW), jnp.float32),
            pltpu.VMEM((R, n_cols, PADW), jnp.float32),
            pltpu.SemaphoreType.DMA,
            pltpu.SemaphoreType.DMA,
            pltpu.SemaphoreType.DMA,
            pltpu.SemaphoreType.DMA,
            pltpu.SemaphoreType.DMA,
            pltpu.SemaphoreType.DMA,
        ],
    )
    def emb_kernel(table_hbm, tok_hbm, out_hbm,
                   idx0, idx1, outb0, outb1, si0, si1, sg0, sg1, so0, so1):
        wid = lax.axis_index("s") * 2 + lax.axis_index("c")
        base_row = wid * rows_per_tec
        idxb = (idx0, idx1)
        outb = (outb0, outb1)
        si = (si0, si1)
        sg = (sg0, sg1)
        so = (so0, so1)

        def run(tbl, tok, out):
            def row0(c):
                return base_row + c * R

            def idx_req(c, b):
                pltpu.async_copy(tok.at[pl.ds(row0(c), R)], idxb[b], si[b])

            def idx_wait(b):
                pltpu.make_async_copy(
                    tok.at[pl.ds(0, R)], idxb[b], si[b]).wait()

            def fire(b):
                @pl.loop(0, R)
                def _(j):
                    for off, w in SPLITS:
                        pltpu.async_copy(
                            tbl.at[idxb[b].at[j, pl.ds(off, w)]],
                            outb[b].at[j, pl.ds(off, w)], sg[b])

            def drain(b):
                @pl.loop(0, R)
                def _(j):
                    for off, w in SPLITS:
                        pltpu.make_async_copy(
                            tbl.at[idxb[b].at[j, pl.ds(off, w)]],
                            outb[b].at[j, pl.ds(off, w)], sg[b]).wait()

            def wb_start(c, b):
                pltpu.async_copy(outb[b].at[:, :, pl.ds(0, DIM)],
                                 out.at[pl.ds(row0(c), R)], so[b])

            def wb_wait(b):
                pltpu.make_async_copy(
                    outb[b].at[:, :, pl.ds(0, DIM)],
                    out.at[pl.ds(0, R)], so[b]).wait()

            idx_req(0, 0)
            idx_req(1, 1)
            idx_wait(0)
            fire(0)

            @pl.loop(0, n_ch // 2)
            def _(k):
                for half in (0, 1):
                    c = 2 * k + half
                    b = half
                    nb = 1 - half
                    @pl.when(c + 1 < n_ch)
                    def _():
                        idx_wait(nb)

                    @pl.when(c >= 1)
                    def _():
                        wb_wait(nb)

                    @pl.when(c + 1 < n_ch)
                    def _():
                        fire(nb)

                    drain(b)

                    @pl.loop(0, R)
                    def _(j):
                        _zero_padding_rows(idxb[b].at[j], outb[b].at[j])

                    @pl.when(c + 2 < n_ch)
                    def _():
                        idx_req(c + 2, b)

                    wb_start(c, b)

            wb_wait((n_ch - 1) % 2)

        run(table_hbm, tok_hbm, out_hbm)

    return emb_kernel


def kernel(src_tokens, tgt_tokens, src_table, tgt_table):
    b, l = src_tokens.shape
    src_idx = src_tokens.astype(jnp.int32)
    tgt_idx = tgt_tokens.astype(jnp.int32)
    src_pad = jnp.pad(src_table, ((0, 0), (0, PADW - DIM)))
    tgt_pad = jnp.pad(tgt_table, ((0, 0), (0, PADW - DIM)))
    emb = _make_kernel(b, l)
    return (emb(src_pad, src_idx), emb(tgt_pad, tgt_idx))


# speedup vs baseline: 1.1030x; 1.1030x over previous
import dataclasses
import functools

import jax
import jax.numpy as jnp
from jax import lax
from jax.experimental import pallas as pl
from jax.experimental.pallas import tpu as pltpu
from jax.experimental.pallas import tpu_sc as plsc

DIM = 32
R = 8
LANES = 16
NTEC = 32
SPLITS = ((0, 128), (128, 72))


def _zero_padding_rows(idx_row, out_row):
    zeros = jnp.zeros((LANES,), jnp.float32)
    for off in list(range(0, 192, LANES)) + [200 - LANES]:
        v = idx_row[pl.ds(off, LANES)]
        is_pad = v == 0

        @pl.when(jnp.any(is_pad))
        def _():
            rows = jnp.arange(LANES, dtype=jnp.int32) + off

            @pl.loop(0, DIM)
            def _(c):
                cols = jnp.full((LANES,), 0, jnp.int32) + c
                plsc.store_scatter(out_row, [rows, cols], zeros, mask=is_pad)


def _make_kernel(n_rows, n_cols):
    mesh = plsc.VectorSubcoreMesh(core_axis_name="c", subcore_axis_name="s")
    out_sds = jax.ShapeDtypeStruct((n_rows, n_cols, DIM), jnp.float32)
    rows_per_tec = n_rows // NTEC
    n_ch = rows_per_tec // R

    cp = pltpu.CompilerParams()
    fields = pltpu.CompilerParams.__dataclass_fields__
    if "needs_layout_passes" in fields:
        cp = dataclasses.replace(cp, needs_layout_passes=False)
    if "use_tc_tiling_on_sc" in fields:
        cp = dataclasses.replace(cp, use_tc_tiling_on_sc=False)

    @functools.partial(
        pl.kernel,
        out_type=out_sds,
        mesh=mesh,
        compiler_params=cp,
        scratch_types=[
            pltpu.VMEM((R, n_cols), jnp.int32),
            pltpu.VMEM((R, n_cols), jnp.int32),
            pltpu.VMEM((R, n_cols, DIM), jnp.float32),
            pltpu.VMEM((R, n_cols, DIM), jnp.float32),
            pltpu.SemaphoreType.DMA,
            pltpu.SemaphoreType.DMA,
            pltpu.SemaphoreType.DMA,
            pltpu.SemaphoreType.DMA,
            pltpu.SemaphoreType.DMA,
            pltpu.SemaphoreType.DMA,
        ],
    )
    def emb_kernel(table_hbm, tok_hbm, out_hbm,
                   idx0, idx1, outb0, outb1, si0, si1, sg0, sg1, so0, so1):
        wid = lax.axis_index("s") * 2 + lax.axis_index("c")
        base_row = wid * rows_per_tec
        idxb = (idx0, idx1)
        outb = (outb0, outb1)
        si = (si0, si1)
        sg = (sg0, sg1)
        so = (so0, so1)

        def run(tbl, tok, out):
            def row0(c):
                return base_row + c * R

            def idx_req(c, b):
                pltpu.async_copy(tok.at[pl.ds(row0(c), R)], idxb[b], si[b])

            def idx_wait(b):
                pltpu.make_async_copy(
                    tok.at[pl.ds(0, R)], idxb[b], si[b]).wait()

            def fire(b):
                @pl.loop(0, R)
                def _(j):
                    for off, w in SPLITS:
                        pltpu.async_copy(
                            tbl.at[idxb[b].at[j, pl.ds(off, w)]],
                            outb[b].at[j, pl.ds(off, w)], sg[b])

            def drain(b):
                @pl.loop(0, R)
                def _(j):
                    for off, w in SPLITS:
                        pltpu.make_async_copy(
                            tbl.at[idxb[b].at[j, pl.ds(off, w)]],
                            outb[b].at[j, pl.ds(off, w)], sg[b]).wait()

            def wb_start(c, b):
                pltpu.async_copy(outb[b], out.at[pl.ds(row0(c), R)], so[b])

            def wb_wait(b):
                pltpu.make_async_copy(
                    outb[b], out.at[pl.ds(0, R)], so[b]).wait()

            idx_req(0, 0)
            idx_req(1, 1)
            idx_wait(0)
            fire(0)

            @pl.loop(0, n_ch // 2)
            def _(k):
                for half in (0, 1):
                    c = 2 * k + half
                    b = half
                    nb = 1 - half
                    @pl.when(c + 1 < n_ch)
                    def _():
                        idx_wait(nb)

                    @pl.when(c >= 1)
                    def _():
                        wb_wait(nb)

                    @pl.when(c + 1 < n_ch)
                    def _():
                        fire(nb)

                    drain(b)

                    @pl.loop(0, R)
                    def _(j):
                        _zero_padding_rows(idxb[b].at[j], outb[b].at[j])

                    @pl.when(c + 2 < n_ch)
                    def _():
                        idx_req(c + 2, b)

                    wb_start(c, b)

            wb_wait((n_ch - 1) % 2)

        run(table_hbm, tok_hbm, out_hbm)

    return emb_kernel


def kernel(src_tokens, tgt_tokens, src_table, tgt_table):
    b, l = src_tokens.shape
    src_idx = src_tokens.astype(jnp.int32)
    tgt_idx = tgt_tokens.astype(jnp.int32)
    emb = _make_kernel(b, l)
    return (emb(src_table, src_idx), emb(tgt_table, tgt_idx))
